# baseline (speedup 1.0000x reference)
import functools
import math

import jax
import jax.numpy as jnp
from jax.experimental import pallas as pl
from jax.experimental.pallas import tpu as pltpu

_BI = 16


def _egcl_kernel(n, bi,
                 x_blk_ref, x_full_ref, h_blk_ref, h_full_ref,
                 We1s_ref, We1r_ref, w0_ref, be1_ref,
                 We2_ref, be2_ref, wi_ref, bi_ref,
                 Wx1_ref, bx1_ref, wxo_ref, bxo_ref,
                 Wh1_ref, bh1_ref, Who_ref, bho_ref,
                 vec_out_ref, feat_out_ref):
    f32 = jnp.float32
    i0 = pl.program_id(0) * bi

    x_blk = x_blk_ref[...]
    h_blk = h_blk_ref[...]
    We2 = We2_ref[...]
    be2 = be2_ref[...]

    diff = x_blk[:, None, :] - x_full_ref[...][None, :, :]
    len2 = jnp.sum(diff * diff, axis=2, keepdims=True) + 1e-16
    lengths = jnp.sqrt(len2)

    A_s = jnp.dot(h_full_ref[...], We1s_ref[...], preferred_element_type=f32)
    A_r = (jnp.dot(h_blk, We1r_ref[...], preferred_element_type=f32)
           + be1_ref[...])
    h1 = (len2 * w0_ref[...][None, :, :] + A_s[None, :, :]
          + A_r[:, None, :])
    s = h1 * jax.nn.sigmoid(h1)

    Wex = jnp.dot(We2, Wx1_ref[...], preferred_element_type=f32)
    Wei = jnp.dot(We2, wi_ref[...], preferred_element_type=f32)
    Wcomb = jnp.concatenate([Wex, Wei], axis=1)
    m_dim = We2.shape[0]
    out = jnp.dot(s.reshape(bi * n, m_dim), Wcomb,
                  preferred_element_type=f32).reshape(bi, n, m_dim + 1)

    cei = jnp.dot(be2, wi_ref[...], preferred_element_type=f32)[0, 0] \
        + bi_ref[0, 0]
    rows = jax.lax.broadcasted_iota(jnp.int32, (bi, n, 1), 0)
    cols = jax.lax.broadcasted_iota(jnp.int32, (bi, n, 1), 1)
    e = jnp.where(cols == rows + i0, 0.0,
                  jax.nn.sigmoid(out[:, :, m_dim:] + cei))
    m_agg = jnp.sum(e * s, axis=1)
    esum = jnp.sum(e, axis=1)
    m_i = jnp.dot(m_agg, We2, preferred_element_type=f32) + esum * be2

    bex = jnp.dot(be2, Wx1_ref[...], preferred_element_type=f32) \
        + bx1_ref[...]
    px1 = out[:, :, :m_dim] + bex[None, :, :]
    px1 = px1 * jax.nn.sigmoid(px1)
    px = (jnp.sum(px1 * wxo_ref[...][None, :, :], axis=2, keepdims=True)
          + bxo_ref[0, 0])
    coef = px / (1.0 + lengths)
    shift = jnp.sum(coef * diff, axis=1)
    vec_out_ref[...] = x_blk + shift * (1.0 / (n - 1))

    m_i_s = m_i * (1.0 / math.sqrt(n - 1.0))
    phin = jnp.concatenate([m_i_s, h_blk], axis=1)
    ph = jnp.dot(phin, Wh1_ref[...], preferred_element_type=f32) + bh1_ref[...]
    feat = jnp.dot(ph, Who_ref[...], preferred_element_type=f32) + bho_ref[...]
    feat_out_ref[...] = feat + h_blk


def kernel(node_vectors, node_features, We1, be1, We2, be2, Wi, bi,
           Wx1, bx1, Wxo, bxo, Wh1, bh1, Who, bho):
    n = node_vectors.shape[0]
    nvec = node_vectors.shape[1]
    h = node_features.shape[1]
    m = We1.shape[1]
    bi_blk = _BI

    x = node_vectors.reshape(n, 3)

    w0 = We1[0:1, :]
    We1s = We1[1:1 + h, :]
    We1r = We1[1 + h:, :]

    full = lambda a: pl.BlockSpec(a.shape, lambda i: (0,) * a.ndim)
    row2 = lambda a: pl.BlockSpec((1, a.shape[-1]), lambda i: (0, 0))

    grid = (n // bi_blk,)
    out_shapes = (
        jax.ShapeDtypeStruct((n, 3), jnp.float32),
        jax.ShapeDtypeStruct((n, h), jnp.float32),
    )
    in_specs = [
        pl.BlockSpec((bi_blk, 3), lambda i: (i, 0)),
        full(x),
        pl.BlockSpec((bi_blk, h), lambda i: (i, 0)),
        full(node_features),
        full(We1s), full(We1r), row2(w0), row2(be1.reshape(1, m)),
        full(We2), row2(be2.reshape(1, h)),
        full(Wi), row2(bi.reshape(1, 1)),
        full(Wx1), row2(bx1.reshape(1, m)),
        row2(Wxo.reshape(1, m)), row2(bxo.reshape(1, 1)),
        full(Wh1), row2(bh1.reshape(1, m)),
        full(Who), row2(bho.reshape(1, h)),
    ]
    out_specs = (
        pl.BlockSpec((bi_blk, 3), lambda i: (i, 0)),
        pl.BlockSpec((bi_blk, h), lambda i: (i, 0)),
    )

    vec, feat = pl.pallas_call(
        functools.partial(_egcl_kernel, n, bi_blk),
        grid=grid,
        in_specs=in_specs,
        out_specs=out_specs,
        out_shape=out_shapes,
        compiler_params=pltpu.CompilerParams(
            dimension_semantics=("parallel",)),
    )(x, x, node_features, node_features,
      We1s, We1r, w0, be1.reshape(1, m),
      We2, be2.reshape(1, h), Wi, bi.reshape(1, 1),
      Wx1, bx1.reshape(1, m), Wxo.reshape(1, m), bxo.reshape(1, 1),
      Wh1, bh1.reshape(1, m), Who, bho.reshape(1, h))

    return vec.reshape(n, nvec, 3), feat


# speedup vs baseline: 1.9987x; 1.9987x over previous
import functools
import math

import jax
import jax.numpy as jnp
from jax.experimental import pallas as pl
from jax.experimental.pallas import tpu as pltpu

_BI = 16


def _egcl_kernel(n, bi,
                 x_blk_ref, x_full_ref, xT_ref, h_blk_ref, h_full_ref,
                 We1s_ref, We1r_ref, w0_ref, be1_ref,
                 We2_ref, be2_ref, wi_ref, bi_ref,
                 Wx1_ref, bx1_ref, wxo_ref, bxo_ref,
                 Wh1_ref, bh1_ref, Who_ref, bho_ref,
                 vec_out_ref, feat_out_ref):
    f32 = jnp.float32
    i0 = pl.program_id(0) * bi

    x_blk = x_blk_ref[...]
    xT = xT_ref[...]
    h_blk = h_blk_ref[...]
    We2 = We2_ref[...]
    be2 = be2_ref[...]
    w0 = w0_ref[...]
    m_dim = We2.shape[0]

    n_row = jnp.sum(xT * xT, axis=0, keepdims=True)
    n_col = jnp.sum(x_blk * x_blk, axis=1, keepdims=True)
    ones_c = jnp.ones((bi, 1), f32)
    Y = jnp.concatenate([x_blk, ones_c, n_col], axis=1)
    XaT = jnp.concatenate([-2.0 * xT, n_row,
                           jnp.ones((1, n), f32)], axis=0)
    len2L = jnp.maximum(jnp.dot(Y, XaT, preferred_element_type=f32),
                        0.0) + 1e-16
    r_inv = 1.0 / (1.0 + jnp.sqrt(len2L))

    A_s = jnp.dot(h_full_ref[...], We1s_ref[...], preferred_element_type=f32)
    A_r = (jnp.dot(h_blk, We1r_ref[...], preferred_element_type=f32)
           + be1_ref[...] + 1e-16 * w0)
    x_full = x_full_ref[...]
    n_colN = jnp.sum(x_full * x_full, axis=1, keepdims=True)
    Xa = jnp.concatenate([-2.0 * x_full, n_colN,
                          jnp.ones((n, 1), f32)], axis=1)
    T1 = Y[:, None, :] * Xa[None, :, :]
    W5 = jnp.broadcast_to(w0, (5, m_dim))
    h1 = (jnp.dot(T1.reshape(bi * n, 5), W5,
                  preferred_element_type=f32).reshape(bi, n, m_dim)
          + A_s[None, :, :] + A_r[:, None, :])
    s = 0.5 * h1 * (1.0 + jnp.tanh(0.5 * h1))

    Wex = jnp.dot(We2, Wx1_ref[...], preferred_element_type=f32)
    Wei = jnp.dot(We2, wi_ref[...], preferred_element_type=f32)
    Wcomb = jnp.concatenate([Wex, Wei], axis=1)
    bex = jnp.dot(be2, Wx1_ref[...], preferred_element_type=f32) \
        + bx1_ref[...]
    cei = jnp.dot(be2, wi_ref[...], preferred_element_type=f32) \
        + bi_ref[...]
    bcomb = jnp.concatenate([bex, cei], axis=1)
    ob = (jnp.dot(s.reshape(bi * n, m_dim), Wcomb,
                  preferred_element_type=f32)
          + bcomb).reshape(bi, n, m_dim + 1)
    T = jnp.tanh(0.5 * ob)

    rows = jax.lax.broadcasted_iota(jnp.int32, (bi, n, 1), 0)
    cols = jax.lax.broadcasted_iota(jnp.int32, (bi, n, 1), 1)
    e = jnp.where(cols == rows + i0, 0.0,
                  0.5 + 0.5 * T[:, :, m_dim:])
    m_agg = jnp.sum(e * s, axis=1)
    esum = jnp.sum(e, axis=1)
    m_i = jnp.dot(m_agg, We2, preferred_element_type=f32) + esum * be2

    px1 = 0.5 * ob[:, :, :m_dim] * (1.0 + T[:, :, :m_dim])
    diffBT = x_blk[:, :, None] - xT[None, :, :]
    rd = r_inv[:, None, :] * diffBT
    PR = jax.lax.dot_general(rd, px1, (((2,), (1,)), ((0,), (0,))),
                             preferred_element_type=f32)
    rdsum = jnp.sum(rd, axis=2)
    shift = (jnp.sum(PR * wxo_ref[...][None, :, :], axis=2)
             + bxo_ref[0, 0] * rdsum)
    vec_out_ref[...] = x_blk + shift * (1.0 / (n - 1))

    m_i_s = m_i * (1.0 / math.sqrt(n - 1.0))
    phin = jnp.concatenate([m_i_s, h_blk], axis=1)
    ph = jnp.dot(phin, Wh1_ref[...], preferred_element_type=f32) + bh1_ref[...]
    feat = jnp.dot(ph, Who_ref[...], preferred_element_type=f32) + bho_ref[...]
    feat_out_ref[...] = feat + h_blk


def kernel(node_vectors, node_features, We1, be1, We2, be2, Wi, bi,
           Wx1, bx1, Wxo, bxo, Wh1, bh1, Who, bho):
    n = node_vectors.shape[0]
    nvec = node_vectors.shape[1]
    h = node_features.shape[1]
    m = We1.shape[1]
    bi_blk = _BI

    x = node_vectors.reshape(n, 3)
    xT = x.T

    w0 = We1[0:1, :]
    We1s = We1[1:1 + h, :]
    We1r = We1[1 + h:, :]

    full = lambda a: pl.BlockSpec(a.shape, lambda i: (0,) * a.ndim)
    row2 = lambda a: pl.BlockSpec((1, a.shape[-1]), lambda i: (0, 0))

    grid = (n // bi_blk,)
    out_shapes = (
        jax.ShapeDtypeStruct((n, 3), jnp.float32),
        jax.ShapeDtypeStruct((n, h), jnp.float32),
    )
    in_specs = [
        pl.BlockSpec((bi_blk, 3), lambda i: (i, 0)),
        full(x),
        full(xT),
        pl.BlockSpec((bi_blk, h), lambda i: (i, 0)),
        full(node_features),
        full(We1s), full(We1r), row2(w0), row2(be1.reshape(1, m)),
        full(We2), row2(be2.reshape(1, h)),
        full(Wi), row2(bi.reshape(1, 1)),
        full(Wx1), row2(bx1.reshape(1, m)),
        row2(Wxo.reshape(1, m)), row2(bxo.reshape(1, 1)),
        full(Wh1), row2(bh1.reshape(1, m)),
        full(Who), row2(bho.reshape(1, h)),
    ]
    out_specs = (
        pl.BlockSpec((bi_blk, 3), lambda i: (i, 0)),
        pl.BlockSpec((bi_blk, h), lambda i: (i, 0)),
    )

    vec, feat = pl.pallas_call(
        functools.partial(_egcl_kernel, n, bi_blk),
        grid=grid,
        in_specs=in_specs,
        out_specs=out_specs,
        out_shape=out_shapes,
        compiler_params=pltpu.CompilerParams(
            dimension_semantics=("parallel",)),
    )(x, x, xT, node_features, node_features,
      We1s, We1r, w0, be1.reshape(1, m),
      We2, be2.reshape(1, h), Wi, bi.reshape(1, 1),
      Wx1, bx1.reshape(1, m), Wxo.reshape(1, m), bxo.reshape(1, 1),
      Wh1, bh1.reshape(1, m), Who, bho.reshape(1, h))

    return vec.reshape(n, nvec, 3), feat


# speedup vs baseline: 2.5900x; 1.2958x over previous
import functools
import math

import jax
import jax.numpy as jnp
from jax.experimental import pallas as pl
from jax.experimental.pallas import tpu as pltpu

_BI = 16


def _egcl_kernel(n, bi,
                 x_blk_ref, x_full_ref, xT_ref, h_blk_ref, h_full_ref,
                 We1s_ref, We1r_ref, w0_ref, be1_ref,
                 We2_ref, be2_ref, wi_ref, bi_ref,
                 Wx1_ref, bx1_ref, wxo_ref, bxo_ref,
                 Wh1_ref, bh1_ref, Who_ref, bho_ref,
                 vec_out_ref, feat_out_ref):
    f32 = jnp.float32
    i0 = pl.program_id(0) * (2 * bi)

    x_blk = x_blk_ref[...]
    xT = xT_ref[...]
    x_full = x_full_ref[...]
    h_blk = h_blk_ref[...]
    We2 = We2_ref[...]
    be2 = be2_ref[...]
    w0 = w0_ref[...]
    m_dim = We2.shape[0]
    zmm = jnp.zeros((m_dim, m_dim), f32)

    n_row = jnp.sum(xT * xT, axis=0, keepdims=True)
    n_col = jnp.sum(x_blk * x_blk, axis=1, keepdims=True)
    Y = jnp.concatenate([x_blk, jnp.ones((2 * bi, 1), f32), n_col],
                        axis=1)
    XaT = jnp.concatenate([-2.0 * xT, n_row,
                           jnp.ones((1, n), f32)], axis=0)
    len2 = jnp.maximum(jnp.dot(Y, XaT, preferred_element_type=f32),
                       0.0) + 1e-16
    r_inv = 1.0 / (1.0 + jnp.sqrt(len2))

    n_colN = jnp.sum(x_full * x_full, axis=1, keepdims=True)
    Xa = jnp.concatenate([-2.0 * x_full, n_colN,
                          jnp.ones((n, 1), f32)], axis=1)
    Xa2 = jnp.concatenate([Xa, Xa], axis=1)
    Y2 = jnp.concatenate([Y[:bi], Y[bi:]], axis=1)
    T1 = Y2[:, None, :] * Xa2[None, :, :]
    W5 = jnp.broadcast_to(w0, (5, m_dim))
    z5 = jnp.zeros((5, m_dim), f32)
    W10 = jnp.concatenate(
        [jnp.concatenate([W5, z5], axis=1),
         jnp.concatenate([z5, W5], axis=1)], axis=0)

    A_s = jnp.dot(h_full_ref[...], We1s_ref[...], preferred_element_type=f32)
    A_s2 = jnp.concatenate([A_s, A_s], axis=1)
    A_r = (jnp.dot(h_blk, We1r_ref[...], preferred_element_type=f32)
           + be1_ref[...] + 1e-16 * w0)
    A_r2 = jnp.concatenate([A_r[:bi], A_r[bi:]], axis=1)

    h1 = (jnp.dot(T1.reshape(bi * n, 10), W10,
                  preferred_element_type=f32).reshape(bi, n, 2 * m_dim)
          + A_s2[None, :, :] + A_r2[:, None, :])
    s = 0.5 * h1 * (1.0 + jnp.tanh(0.5 * h1))

    Wex = jnp.dot(We2, Wx1_ref[...], preferred_element_type=f32)
    Wexbd = jnp.concatenate(
        [jnp.concatenate([Wex, zmm], axis=1),
         jnp.concatenate([zmm, Wex], axis=1)], axis=0)
    bex = jnp.dot(be2, Wx1_ref[...], preferred_element_type=f32) \
        + bx1_ref[...]
    bex2 = jnp.concatenate([bex, bex], axis=1)
    sr = s.reshape(bi * n, 2 * m_dim)
    ob = (jnp.dot(sr, Wexbd, preferred_element_type=f32)
          + bex2).reshape(bi, n, 2 * m_dim)
    T = jnp.tanh(0.5 * ob)
    px1 = 0.5 * ob * (1.0 + T)

    Wei = jnp.dot(We2, wi_ref[...], preferred_element_type=f32)
    zm1 = jnp.zeros((m_dim, 1), f32)
    Wei2 = jnp.concatenate(
        [jnp.concatenate([Wei, zm1], axis=1),
         jnp.concatenate([zm1, Wei], axis=1)], axis=0)
    cei = jnp.dot(be2, wi_ref[...], preferred_element_type=f32)[0, 0] \
        + bi_ref[0, 0]
    elog = (jnp.dot(sr, Wei2, preferred_element_type=f32)
            .reshape(bi, n, 2) + cei)
    rows = jax.lax.broadcasted_iota(jnp.int32, (bi, n, 2), 0)
    cols = jax.lax.broadcasted_iota(jnp.int32, (bi, n, 2), 1)
    half = jax.lax.broadcasted_iota(jnp.int32, (bi, n, 2), 2)
    e = jnp.where(cols == i0 + rows + bi * half, 0.0,
                  0.5 + 0.5 * jnp.tanh(0.5 * elog))
    e_a = e[:, :, 0:1]
    e_b = e[:, :, 1:2]
    mA = jax.lax.dot_general(e_a, s, (((1,), (1,)), ((0,), (0,))),
                             preferred_element_type=f32)
    mB = jax.lax.dot_general(e_b, s, (((1,), (1,)), ((0,), (0,))),
                             preferred_element_type=f32)
    m_agg = jnp.concatenate([mA[:, 0, :m_dim], mB[:, 0, m_dim:]],
                            axis=0)
    esum2 = jnp.sum(e, axis=1)
    esum = jnp.concatenate([esum2[:, 0:1], esum2[:, 1:2]], axis=0)
    m_i = jnp.dot(m_agg, We2, preferred_element_type=f32) + esum * be2

    diffBT = x_blk[:, :, None] - xT[None, :, :]
    rd = r_inv[:, None, :] * diffBT
    PR_a = jax.lax.dot_general(rd[:bi], px1[:, :, :m_dim],
                               (((2,), (1,)), ((0,), (0,))),
                               preferred_element_type=f32)
    PR_b = jax.lax.dot_general(rd[bi:], px1[:, :, m_dim:],
                               (((2,), (1,)), ((0,), (0,))),
                               preferred_element_type=f32)
    PR = jnp.concatenate([PR_a, PR_b], axis=0)
    rdsum = jnp.sum(rd, axis=2)
    shift = (jnp.sum(PR * wxo_ref[...][None, :, :], axis=2)
             + bxo_ref[0, 0] * rdsum)
    vec_out_ref[...] = x_blk + shift * (1.0 / (n - 1))

    m_i_s = m_i * (1.0 / math.sqrt(n - 1.0))
    phin = jnp.concatenate([m_i_s, h_blk], axis=1)
    ph = jnp.dot(phin, Wh1_ref[...], preferred_element_type=f32) + bh1_ref[...]
    feat = jnp.dot(ph, Who_ref[...], preferred_element_type=f32) + bho_ref[...]
    feat_out_ref[...] = feat + h_blk


def kernel(node_vectors, node_features, We1, be1, We2, be2, Wi, bi,
           Wx1, bx1, Wxo, bxo, Wh1, bh1, Who, bho):
    n = node_vectors.shape[0]
    nvec = node_vectors.shape[1]
    h = node_features.shape[1]
    m = We1.shape[1]
    bi_blk = _BI
    rows_blk = 2 * bi_blk

    x = node_vectors.reshape(n, 3)
    xT = x.T

    w0 = We1[0:1, :]
    We1s = We1[1:1 + h, :]
    We1r = We1[1 + h:, :]

    full = lambda a: pl.BlockSpec(a.shape, lambda i: (0,) * a.ndim)
    row2 = lambda a: pl.BlockSpec((1, a.shape[-1]), lambda i: (0, 0))

    grid = (n // rows_blk,)
    out_shapes = (
        jax.ShapeDtypeStruct((n, 3), jnp.float32),
        jax.ShapeDtypeStruct((n, h), jnp.float32),
    )
    in_specs = [
        pl.BlockSpec((rows_blk, 3), lambda i: (i, 0)),
        full(x),
        full(xT),
        pl.BlockSpec((rows_blk, h), lambda i: (i, 0)),
        full(node_features),
        full(We1s), full(We1r), row2(w0), row2(be1.reshape(1, m)),
        full(We2), row2(be2.reshape(1, h)),
        full(Wi), row2(bi.reshape(1, 1)),
        full(Wx1), row2(bx1.reshape(1, m)),
        row2(Wxo.reshape(1, m)), row2(bxo.reshape(1, 1)),
        full(Wh1), row2(bh1.reshape(1, m)),
        full(Who), row2(bho.reshape(1, h)),
    ]
    out_specs = (
        pl.BlockSpec((rows_blk, 3), lambda i: (i, 0)),
        pl.BlockSpec((rows_blk, h), lambda i: (i, 0)),
    )

    vec, feat = pl.pallas_call(
        functools.partial(_egcl_kernel, n, bi_blk),
        grid=grid,
        in_specs=in_specs,
        out_specs=out_specs,
        out_shape=out_shapes,
        compiler_params=pltpu.CompilerParams(
            dimension_semantics=("parallel",)),
    )(x, x, xT, node_features, node_features,
      We1s, We1r, w0, be1.reshape(1, m),
      We2, be2.reshape(1, h), Wi, bi.reshape(1, 1),
      Wx1, bx1.reshape(1, m), Wxo.reshape(1, m), bxo.reshape(1, 1),
      Wh1, bh1.reshape(1, m), Who, bho.reshape(1, h))

    return vec.reshape(n, nvec, 3), feat


# speedup vs baseline: 3.2624x; 1.2596x over previous
import functools
import math

import jax
import jax.numpy as jnp
from jax.experimental import pallas as pl
from jax.experimental.pallas import tpu as pltpu

_BI = 16


def _egcl_kernel(n, bi,
                 x_blk_ref, x_full_ref, xT_ref, h_blk_ref, h_full_ref,
                 We1s_ref, We1r_ref, w0_ref, be1_ref,
                 We2_ref, be2_ref, wi_ref, bi_ref,
                 Wx1_ref, bx1_ref, wxo_ref, bxo_ref,
                 Wh1_ref, bh1_ref, Who_ref, bho_ref,
                 vec_out_ref, feat_out_ref):
    f32 = jnp.float32
    i0 = pl.program_id(0) * (2 * bi)

    x_blk = x_blk_ref[...]
    xT = xT_ref[...]
    x_full = x_full_ref[...]
    h_blk = h_blk_ref[...]
    We2 = We2_ref[...]
    be2 = be2_ref[...]
    w0 = w0_ref[...]
    m_dim = We2.shape[0]
    zmm = jnp.zeros((m_dim, m_dim), f32)

    n_row = jnp.sum(xT * xT, axis=0, keepdims=True)
    n_col = jnp.sum(x_blk * x_blk, axis=1, keepdims=True)
    Y = jnp.concatenate([x_blk, jnp.ones((2 * bi, 1), f32), n_col],
                        axis=1)
    XaT = jnp.concatenate([-2.0 * xT, n_row,
                           jnp.ones((1, n), f32)], axis=0)
    len2 = jnp.maximum(jnp.dot(Y, XaT, preferred_element_type=f32),
                       0.0) + 1e-16
    r_inv = 1.0 / (1.0 + jnp.sqrt(len2))

    n_colN = jnp.sum(x_full * x_full, axis=1, keepdims=True)
    Xa = jnp.concatenate([-2.0 * x_full, n_colN,
                          jnp.ones((n, 1), f32)], axis=1)
    Xa2 = jnp.concatenate([Xa, Xa], axis=1)
    Y2 = jnp.concatenate([Y[:bi], Y[bi:]], axis=1)
    T1 = Y2[:, None, :] * Xa2[None, :, :]
    W5 = jnp.broadcast_to(w0, (5, m_dim))
    z5 = jnp.zeros((5, m_dim), f32)
    W10 = jnp.concatenate(
        [jnp.concatenate([W5, z5], axis=1),
         jnp.concatenate([z5, W5], axis=1)], axis=0)

    A_s = jnp.dot(h_full_ref[...], We1s_ref[...], preferred_element_type=f32)
    A_s2 = jnp.concatenate([A_s, A_s], axis=1)
    A_r = (jnp.dot(h_blk, We1r_ref[...], preferred_element_type=f32)
           + be1_ref[...] + 1e-16 * w0)
    A_r2 = jnp.concatenate([A_r[:bi], A_r[bi:]], axis=1)

    h1 = (jnp.dot(T1.reshape(bi * n, 10), W10,
                  preferred_element_type=f32).reshape(bi, n, 2 * m_dim)
          + A_s2[None, :, :] + A_r2[:, None, :])
    s = 0.5 * h1 * (1.0 + jnp.tanh(0.5 * h1))

    Wex = jnp.dot(We2, Wx1_ref[...], preferred_element_type=f32)
    Wexbd = jnp.concatenate(
        [jnp.concatenate([Wex, zmm], axis=1),
         jnp.concatenate([zmm, Wex], axis=1)], axis=0)
    bex = jnp.dot(be2, Wx1_ref[...], preferred_element_type=f32) \
        + bx1_ref[...]
    bex2 = jnp.concatenate([bex, bex], axis=1)
    sr = s.reshape(bi * n, 2 * m_dim)
    ob = (jnp.dot(sr, Wexbd, preferred_element_type=f32)
          + bex2).reshape(bi, n, 2 * m_dim)
    T = jnp.tanh(0.5 * ob)
    px1 = 0.5 * ob * (1.0 + T)

    Wei = jnp.dot(We2, wi_ref[...], preferred_element_type=f32)
    zm1 = jnp.zeros((m_dim, 1), f32)
    Wei2T = jnp.concatenate(
        [jnp.concatenate([Wei, zm1], axis=0),
         jnp.concatenate([zm1, Wei], axis=0)], axis=1).T
    cei = jnp.dot(be2, wi_ref[...], preferred_element_type=f32)[0, 0] \
        + bi_ref[0, 0]
    elogT = jax.lax.dot_general(Wei2T, s, (((1,), (2,)), ((), ())),
                                preferred_element_type=f32) + cei
    half = jax.lax.broadcasted_iota(jnp.int32, (2, bi, n), 0)
    rows = jax.lax.broadcasted_iota(jnp.int32, (2, bi, n), 1)
    cols = jax.lax.broadcasted_iota(jnp.int32, (2, bi, n), 2)
    eT = jnp.where(cols == i0 + rows + bi * half, 0.0,
                   0.5 + 0.5 * jnp.tanh(0.5 * elogT))
    mA = jax.lax.dot_general(eT[0], s, (((1,), (1,)), ((0,), (0,))),
                             preferred_element_type=f32)
    mB = jax.lax.dot_general(eT[1], s, (((1,), (1,)), ((0,), (0,))),
                             preferred_element_type=f32)
    m_agg = jnp.concatenate([mA[:, :m_dim], mB[:, m_dim:]],
                            axis=0)
    esum = jnp.sum(eT, axis=2).reshape(2 * bi, 1)
    m_i = jnp.dot(m_agg, We2, preferred_element_type=f32) + esum * be2

    diffBT = x_blk[:, :, None] - xT[None, :, :]
    rd = r_inv[:, None, :] * diffBT
    PR_a = jax.lax.dot_general(rd[:bi], px1[:, :, :m_dim],
                               (((2,), (1,)), ((0,), (0,))),
                               preferred_element_type=f32)
    PR_b = jax.lax.dot_general(rd[bi:], px1[:, :, m_dim:],
                               (((2,), (1,)), ((0,), (0,))),
                               preferred_element_type=f32)
    PR = jnp.concatenate([PR_a, PR_b], axis=0)
    rdsum = jnp.sum(rd, axis=2)
    shift = (jnp.sum(PR * wxo_ref[...][None, :, :], axis=2)
             + bxo_ref[0, 0] * rdsum)
    vec_out_ref[...] = x_blk + shift * (1.0 / (n - 1))

    m_i_s = m_i * (1.0 / math.sqrt(n - 1.0))
    phin = jnp.concatenate([m_i_s, h_blk], axis=1)
    ph = jnp.dot(phin, Wh1_ref[...], preferred_element_type=f32) + bh1_ref[...]
    feat = jnp.dot(ph, Who_ref[...], preferred_element_type=f32) + bho_ref[...]
    feat_out_ref[...] = feat + h_blk


def kernel(node_vectors, node_features, We1, be1, We2, be2, Wi, bi,
           Wx1, bx1, Wxo, bxo, Wh1, bh1, Who, bho):
    n = node_vectors.shape[0]
    nvec = node_vectors.shape[1]
    h = node_features.shape[1]
    m = We1.shape[1]
    bi_blk = _BI
    rows_blk = 2 * bi_blk

    x = node_vectors.reshape(n, 3)
    xT = x.T

    w0 = We1[0:1, :]
    We1s = We1[1:1 + h, :]
    We1r = We1[1 + h:, :]

    full = lambda a: pl.BlockSpec(a.shape, lambda i: (0,) * a.ndim)
    row2 = lambda a: pl.BlockSpec((1, a.shape[-1]), lambda i: (0, 0))

    grid = (n // rows_blk,)
    out_shapes = (
        jax.ShapeDtypeStruct((n, 3), jnp.float32),
        jax.ShapeDtypeStruct((n, h), jnp.float32),
    )
    in_specs = [
        pl.BlockSpec((rows_blk, 3), lambda i: (i, 0)),
        full(x),
        full(xT),
        pl.BlockSpec((rows_blk, h), lambda i: (i, 0)),
        full(node_features),
        full(We1s), full(We1r), row2(w0), row2(be1.reshape(1, m)),
        full(We2), row2(be2.reshape(1, h)),
        full(Wi), row2(bi.reshape(1, 1)),
        full(Wx1), row2(bx1.reshape(1, m)),
        row2(Wxo.reshape(1, m)), row2(bxo.reshape(1, 1)),
        full(Wh1), row2(bh1.reshape(1, m)),
        full(Who), row2(bho.reshape(1, h)),
    ]
    out_specs = (
        pl.BlockSpec((rows_blk, 3), lambda i: (i, 0)),
        pl.BlockSpec((rows_blk, h), lambda i: (i, 0)),
    )

    vec, feat = pl.pallas_call(
        functools.partial(_egcl_kernel, n, bi_blk),
        grid=grid,
        in_specs=in_specs,
        out_specs=out_specs,
        out_shape=out_shapes,
        compiler_params=pltpu.CompilerParams(
            dimension_semantics=("parallel",)),
    )(x, x, xT, node_features, node_features,
      We1s, We1r, w0, be1.reshape(1, m),
      We2, be2.reshape(1, h), Wi, bi.reshape(1, 1),
      Wx1, bx1.reshape(1, m), Wxo.reshape(1, m), bxo.reshape(1, 1),
      Wh1, bh1.reshape(1, m), Who, bho.reshape(1, h))

    return vec.reshape(n, nvec, 3), feat
